# Initial kernel scaffold; baseline (speedup 1.0000x reference)
import jax
import jax.numpy as jnp
from jax.experimental import pallas as pl


def kernel(input, embedding_weight):
    raise NotImplementedError("write your pallas kernel here")


# speedup vs baseline: 4.0830x; 4.0830x over previous
import functools

import jax
import jax.numpy as jnp
from jax import lax
from jax.experimental import pallas as pl
from jax.experimental.pallas import tpu as pltpu
from jax.experimental.pallas import tpu_sc as plsc

N_V = 100000
N_D = 64
N_B = 4096
N_S = 50

NC, NS = 2, 16
NW = NC * NS
B = N_B * N_S
BPW = B // NW
C = 128
NCH = BPW // C

_mesh = plsc.VectorSubcoreMesh(
    core_axis_name="c", subcore_axis_name="s", num_cores=NC, num_subcores=NS
)


@functools.partial(
    pl.kernel,
    out_type=jax.ShapeDtypeStruct((B, N_D), jnp.float32),
    mesh=_mesh,
    scratch_types=[
        pltpu.VMEM((NCH, C), jnp.int32),
        pltpu.VMEM((2, C, N_D), jnp.float32),
        pltpu.SemaphoreType.DMA,
    ],
    compiler_params=pltpu.CompilerParams(use_tc_tiling_on_sc=False),
)
def _embed_gather(idx_hbm, table_hbm, out_hbm, idx_v, rows_v, gsem):
    wid = lax.axis_index("s") * NC + lax.axis_index("c")
    base = wid * BPW
    pltpu.sync_copy(idx_hbm.at[wid], idx_v)

    @pl.loop(0, NCH)
    def _chunk(i):
        pltpu.async_copy(table_hbm.at[idx_v.at[i]], rows_v.at[0], gsem).wait()
        pltpu.sync_copy(rows_v.at[0], out_hbm.at[pl.ds(base + i * C, C)])


def kernel(input, embedding_weight):
    idx = input.reshape(-1).astype(jnp.int32).reshape(NW, NCH, C)
    out = _embed_gather(idx, embedding_weight)
    return out.reshape(N_B, N_S, N_D)


# speedup vs baseline: 4.6308x; 1.1342x over previous
import functools

import jax
import jax.numpy as jnp
from jax import lax
from jax.experimental import pallas as pl
from jax.experimental.pallas import tpu as pltpu
from jax.experimental.pallas import tpu_sc as plsc

N_V = 100000
N_D = 64
N_B = 4096
N_S = 50

NC, NS = 2, 16
NW = NC * NS
B = N_B * N_S
BPW = B // NW
C = 128
NCH = BPW // C
NBUF = 5
GC = NBUF * C
NR = BPW // GC

_mesh = plsc.VectorSubcoreMesh(
    core_axis_name="c", subcore_axis_name="s", num_cores=NC, num_subcores=NS
)


@functools.partial(
    pl.kernel,
    out_type=jax.ShapeDtypeStruct((B, N_D), jnp.float32),
    mesh=_mesh,
    scratch_types=[
        pltpu.VMEM((NCH, C), jnp.int32),
        pltpu.VMEM((2, GC, N_D), jnp.float32),
        pltpu.SemaphoreType.DMA,
        pltpu.SemaphoreType.DMA,
        pltpu.SemaphoreType.DMA,
        pltpu.SemaphoreType.DMA,
    ],
    compiler_params=pltpu.CompilerParams(use_tc_tiling_on_sc=False),
)
def _embed_gather(idx_hbm, table_hbm, out_hbm, idx_v, rows_v, g0, g1, o0, o1):
    gsems = (g0, g1)
    osems = (o0, o1)
    wid = lax.axis_index("s") * NC + lax.axis_index("c")
    base = wid * BPW
    pltpu.sync_copy(idx_hbm.at[wid], idx_v)

    def fire(r, g):
        for k in range(NBUF):
            pltpu.async_copy(
                table_hbm.at[idx_v.at[r * NBUF + k]],
                rows_v.at[g, pl.ds(k * C, C)],
                gsems[g],
            )

    def drain_gather(g):
        pltpu.make_async_copy(
            out_hbm.at[pl.ds(base, GC)], rows_v.at[g], gsems[g]
        ).wait()

    def write(r, g):
        pltpu.async_copy(
            rows_v.at[g], out_hbm.at[pl.ds(base + r * GC, GC)], osems[g]
        )

    def drain_write(g):
        pltpu.make_async_copy(
            rows_v.at[g], out_hbm.at[pl.ds(base, GC)], osems[g]
        ).wait()

    fire(0, 0)
    fire(1, 1)
    drain_gather(0)
    write(0, 0)

    @pl.loop(1, NR - 1, step=2)
    def _steady(r0):
        for b in range(2):
            r = r0 + b
            g = 1 - b
            og = b
            drain_gather(g)
            drain_write(og)
            fire(r + 1, og)
            write(r, g)

    drain_gather(1)
    drain_write(0)
    write(NR - 1, 1)
    drain_write(1)


def kernel(input, embedding_weight):
    idx = input.reshape(-1).astype(jnp.int32).reshape(NW, NCH, C)
    out = _embed_gather(idx, embedding_weight)
    return out.reshape(N_B, N_S, N_D)


# speedup vs baseline: 4.6432x; 1.0027x over previous
import functools

import jax
import jax.numpy as jnp
from jax import lax
from jax.experimental import pallas as pl
from jax.experimental.pallas import tpu as pltpu
from jax.experimental.pallas import tpu_sc as plsc

N_V = 100000
N_D = 64
N_B = 4096
N_S = 50

NC, NS = 2, 16
NW = NC * NS
B = N_B * N_S
BPW = B // NW
C = 640
NCH = BPW // C
NBUF = 1
GC = NBUF * C
NR = BPW // GC

_mesh = plsc.VectorSubcoreMesh(
    core_axis_name="c", subcore_axis_name="s", num_cores=NC, num_subcores=NS
)


@functools.partial(
    pl.kernel,
    out_type=jax.ShapeDtypeStruct((B, N_D), jnp.float32),
    mesh=_mesh,
    scratch_types=[
        pltpu.VMEM((NCH, C), jnp.int32),
        pltpu.VMEM((2, GC, N_D), jnp.float32),
        pltpu.SemaphoreType.DMA,
        pltpu.SemaphoreType.DMA,
        pltpu.SemaphoreType.DMA,
        pltpu.SemaphoreType.DMA,
    ],
    compiler_params=pltpu.CompilerParams(use_tc_tiling_on_sc=False),
)
def _embed_gather(idx_hbm, table_hbm, out_hbm, idx_v, rows_v, g0, g1, o0, o1):
    gsems = (g0, g1)
    osems = (o0, o1)
    wid = lax.axis_index("s") * NC + lax.axis_index("c")
    base = wid * BPW
    pltpu.sync_copy(idx_hbm.at[wid], idx_v)

    def fire(r, g):
        for k in range(NBUF):
            pltpu.async_copy(
                table_hbm.at[idx_v.at[r * NBUF + k]],
                rows_v.at[g, pl.ds(k * C, C)],
                gsems[g],
            )

    def drain_gather(g):
        pltpu.make_async_copy(
            out_hbm.at[pl.ds(base, GC)], rows_v.at[g], gsems[g]
        ).wait()

    def write(r, g):
        pltpu.async_copy(
            rows_v.at[g], out_hbm.at[pl.ds(base + r * GC, GC)], osems[g]
        )

    def drain_write(g):
        pltpu.make_async_copy(
            rows_v.at[g], out_hbm.at[pl.ds(base, GC)], osems[g]
        ).wait()

    fire(0, 0)
    fire(1, 1)
    drain_gather(0)
    write(0, 0)

    @pl.loop(1, NR - 1, step=2)
    def _steady(r0):
        for b in range(2):
            r = r0 + b
            g = 1 - b
            og = b
            drain_gather(g)
            drain_write(og)
            fire(r + 1, og)
            write(r, g)

    drain_gather(1)
    drain_write(0)
    write(NR - 1, 1)
    drain_write(1)


def kernel(input, embedding_weight):
    idx = input.reshape(-1).astype(jnp.int32).reshape(NW, NCH, C)
    out = _embed_gather(idx, embedding_weight)
    return out.reshape(N_B, N_S, N_D)
